# baseline (speedup 1.0000x reference)
import functools

import jax
import jax.numpy as jnp
from jax import lax
from jax.experimental import pallas as pl
from jax.experimental.pallas import tpu as pltpu
from jax.experimental.pallas import tpu_sc as plsc

N = 10000
E = 320000
EP = 200000
D = 128
H = 256
R = 400
EPP = 204800
RC = 2048

_SQRT_HALF = 0.7071067811865476


def _matT(a, b):
    return lax.dot_general(a, b, (((1,), (1,)), ((), ())),
                           preferred_element_type=jnp.float32)


def _gelu(v):
    return 0.5 * v * (1.0 + lax.erf(v * _SQRT_HALF))


def _a0_body(msum_ref, deg_ref, x_ref, wl_ref, bl_ref, wr_ref, y_ref, st_ref):
    i = pl.program_id(0)
    rdc = 1.0 / jnp.maximum(deg_ref[0, :, 0:1] + deg_ref[1, :, 0:1], 1.0)
    mean = (msum_ref[0] + msum_ref[1]) * rdc
    y = _matT(mean, wl_ref[...]) + bl_ref[...] + _matT(x_ref[...], wr_ref[...])
    y = _gelu(y)
    y_ref[...] = y

    @pl.when(i == 0)
    def _():
        st_ref[...] = jnp.zeros_like(st_ref)

    st_ref[0:1, :] += jnp.sum(y, axis=0, keepdims=True)
    st_ref[1:2, :] += jnp.sum(y * y, axis=0, keepdims=True)


def _a1_body(msum_ref, deg_ref, h_ref, wla_ref, wlb_ref, bl_ref, wra_ref,
             wrb_ref, y_ref, st_ref):
    i = pl.program_id(0)
    rdc = 1.0 / jnp.maximum(deg_ref[0, :, 0:1] + deg_ref[1, :, 0:1], 1.0)
    y = (_matT(msum_ref[0] * rdc, wla_ref[...])
         + _matT(msum_ref[1] * rdc, wlb_ref[...])
         + bl_ref[...]
         + _matT(h_ref[0], wra_ref[...])
         + _matT(h_ref[1], wrb_ref[...]))
    y = _gelu(y)
    y_ref[...] = y

    @pl.when(i == 0)
    def _():
        st_ref[...] = jnp.zeros_like(st_ref)

    st_ref[0:1, :] += jnp.sum(y, axis=0, keepdims=True)
    st_ref[1:2, :] += jnp.sum(y * y, axis=0, keepdims=True)


def _norm0_body(y_ref, st_ref, g_ref, be_ref, ms_ref, h_ref):
    mu = st_ref[0:1, :] * (1.0 / N)
    m2 = st_ref[1:2, :] * (1.0 / N)
    c = mu * ms_ref[...]
    var = m2 - 2.0 * c * mu + c * c
    scale = g_ref[...] * lax.rsqrt(var + 1e-5)
    h = (y_ref[...] - c) * scale + be_ref[...]
    h_ref[0] = h[:, :D]
    h_ref[1] = h[:, D:]


def _norm_res_body(y_ref, st_ref, g_ref, be_ref, ms_ref, r_ref, h_ref):
    mu = st_ref[0:1, :] * (1.0 / N)
    m2 = st_ref[1:2, :] * (1.0 / N)
    c = mu * ms_ref[...]
    var = m2 - 2.0 * c * mu + c * c
    scale = g_ref[...] * lax.rsqrt(var + 1e-5)
    h = (y_ref[...] - c) * scale + be_ref[...]
    h_ref[0] = h[:, :D] + r_ref[0]
    h_ref[1] = h[:, D:] + r_ref[1]


def _z_body(y2_ref, st_ref, g_ref, be_ref, ms_ref, h1_ref, h0_ref, x_ref,
            alpha_ref, dw_ref, db_ref, w1_ref, b1_ref, w2_ref, b2_ref,
            w3_ref, b3_ref, z_ref):
    mu = st_ref[0:1, :] * (1.0 / N)
    m2 = st_ref[1:2, :] * (1.0 / N)
    c = mu * ms_ref[...]
    var = m2 - 2.0 * c * mu + c * c
    scale = g_ref[...] * lax.rsqrt(var + 1e-5)
    h1 = jnp.concatenate([h1_ref[0], h1_ref[1]], axis=1)
    h0 = jnp.concatenate([h0_ref[0], h0_ref[1]], axis=1)
    h2 = (y2_ref[...] - c) * scale + be_ref[...] + h1
    z = jnp.concatenate([h0, h1, h2], axis=1)
    z = jnp.tanh(alpha_ref[0, 0] * z) * dw_ref[...] + db_ref[...]
    t = _gelu(_matT(z, w1_ref[...]) + b1_ref[...])
    t = _gelu(_matT(t, w2_ref[...]) + b2_ref[...])
    t = _matT(t, w3_ref[...]) + b3_ref[...]
    t = t + x_ref[...]
    nrm = jnp.sqrt(jnp.sum(t * t, axis=1, keepdims=True))
    z_ref[...] = t / (nrm + 1e-10)


def _head_body(zi_ref, zj_ref, c1a_ref, c1b_ref, cb1_ref, c2_ref, cb2_ref,
               c3_ref, cb3_ref, o_ref):
    w = _gelu(_matT(zi_ref[...], c1a_ref[...]) + _matT(zj_ref[...], c1b_ref[...])
              + cb1_ref[...])
    w = _gelu(_matT(w, c2_ref[...]) + cb2_ref[...])
    s = jnp.sum(w * c3_ref[...], axis=1) + cb3_ref[0, 0]
    o_ref[...] = 1.0 / (1.0 + jnp.exp(-s))


_GRID = N // R


def _spec(shape, idx):
    return pl.BlockSpec(shape, idx)


def _row_specs():
    return dict(
        msum=_spec((2, R, D), lambda i: (0, i, 0)),
        deg=_spec((2, R, D), lambda i: (0, i, 0)),
        xrow=_spec((R, D), lambda i: (i, 0)),
        hrow=_spec((2, R, D), lambda i: (0, i, 0)),
        yrow=_spec((R, H), lambda i: (i, 0)),
        st=_spec((8, H), lambda i: (0, 0)),
    )


def _layer0(msum, deg, x, Wl0, bl0, Wr0):
    s = _row_specs()
    y, st = pl.pallas_call(
        _a0_body,
        grid=(_GRID,),
        in_specs=[s["msum"], s["deg"], s["xrow"],
                  _spec((H, D), lambda i: (0, 0)),
                  _spec((1, H), lambda i: (0, 0)),
                  _spec((H, D), lambda i: (0, 0))],
        out_specs=[s["yrow"], s["st"]],
        out_shape=[jax.ShapeDtypeStruct((N, H), jnp.float32),
                   jax.ShapeDtypeStruct((8, H), jnp.float32)],
    )(msum, deg, x, Wl0, bl0.reshape(1, H), Wr0)
    return y, st


def _layer(msum, deg, h, Wl, bl, Wr):
    s = _row_specs()
    wspec = _spec((H, D), lambda i: (0, 0))
    y, st = pl.pallas_call(
        _a1_body,
        grid=(_GRID,),
        in_specs=[s["msum"], s["deg"], s["hrow"], wspec, wspec,
                  _spec((1, H), lambda i: (0, 0)), wspec, wspec],
        out_specs=[s["yrow"], s["st"]],
        out_shape=[jax.ShapeDtypeStruct((N, H), jnp.float32),
                   jax.ShapeDtypeStruct((8, H), jnp.float32)],
    )(msum, deg, h, Wl[:, :D], Wl[:, D:], bl.reshape(1, H),
      Wr[:, :D], Wr[:, D:])
    return y, st


def _norm(y, st, g, be, ms, res=None):
    s = _row_specs()
    vspec = _spec((1, H), lambda i: (0, 0))
    in_specs = [s["yrow"], s["st"], vspec, vspec, vspec]
    args = [y, st, g.reshape(1, H), be.reshape(1, H), ms.reshape(1, H)]
    body = _norm0_body
    if res is not None:
        in_specs.append(s["hrow"])
        args.append(res)
        body = _norm_res_body
    return pl.pallas_call(
        body,
        grid=(_GRID,),
        in_specs=in_specs,
        out_specs=s["hrow"],
        out_shape=jax.ShapeDtypeStruct((2, _NPAD, D), jnp.float32),
    )(*args)


def _zstage(y2, st2, g2, be2, ms2, h1, h0, x, alpha, dw, db, W1, b1, W2, b2,
            W3, b3):
    s = _row_specs()
    vspec = _spec((1, H), lambda i: (0, 0))
    JK = 3 * H
    z = pl.pallas_call(
        _z_body,
        grid=(_GRID,),
        in_specs=[s["yrow"], s["st"], vspec, vspec, vspec, s["hrow"],
                  s["hrow"], s["xrow"],
                  _spec((1, 1), lambda i: (0, 0)),
                  _spec((1, JK), lambda i: (0, 0)),
                  _spec((1, JK), lambda i: (0, 0)),
                  _spec((H, JK), lambda i: (0, 0)),
                  _spec((1, H), lambda i: (0, 0)),
                  _spec((H, H), lambda i: (0, 0)),
                  _spec((1, H), lambda i: (0, 0)),
                  _spec((D, H), lambda i: (0, 0)),
                  _spec((1, D), lambda i: (0, 0))],
        out_specs=_spec((R, D), lambda i: (i, 0)),
        out_shape=jax.ShapeDtypeStruct((N, D), jnp.float32),
    )(y2, st2, g2.reshape(1, H), be2.reshape(1, H), ms2.reshape(1, H),
      h1, h0, x, alpha.reshape(1, 1), dw.reshape(1, JK), db.reshape(1, JK),
      W1, b1.reshape(1, H), W2, b2.reshape(1, H), W3, b3.reshape(1, D))
    return z


def _head(zi, zj, C1, cb1, C2, cb2, C3, cb3):
    return pl.pallas_call(
        _head_body,
        grid=(EPP // RC,),
        in_specs=[_spec((RC, D), lambda i: (i, 0)),
                  _spec((RC, D), lambda i: (i, 0)),
                  _spec((64, D), lambda i: (0, 0)),
                  _spec((64, D), lambda i: (0, 0)),
                  _spec((1, 64), lambda i: (0, 0)),
                  _spec((64, 64), lambda i: (0, 0)),
                  _spec((1, 64), lambda i: (0, 0)),
                  _spec((1, 64), lambda i: (0, 0)),
                  _spec((1, 1), lambda i: (0, 0))],
        out_specs=_spec((RC,), lambda i: (i,)),
        out_shape=jax.ShapeDtypeStruct((EPP,), jnp.float32),
    )(zi, zj, C1[:, :D], C1[:, D:], cb1.reshape(1, 64), C2,
      cb2.reshape(1, 64), C3.reshape(1, 64), cb3.reshape(1, 1))


_NC = 2
_NS = 16
_TILES = _NC * _NS
_KE = 125
_KC = 80
_BI = 16
_NPAD = 10240
_RPT = _NPAD // _NS


def _sc_mesh():
    return plsc.VectorSubcoreMesh(core_axis_name="c", subcore_axis_name="s")


def _trows(sid):
    return pl.ds(pl.multiple_of(sid * _RPT, 8), _RPT)


def _msum_pipeline(table_hbm, accs, srcv2, dstv2, rows0, rows1, sem0, sem1,
                   nch):
    pltpu.async_copy(table_hbm.at[srcv2.at[0]], rows0, sem0)
    pltpu.async_copy(table_hbm.at[srcv2.at[1]], rows1, sem1)

    def body(t, carry):
        g0 = t * 2
        g1 = g0 + 1
        pltpu.make_async_copy(table_hbm.at[srcv2.at[g0]], rows0, sem0).wait()
        pltpu.sync_copy(rows0, accs.at[dstv2.at[g0]], add=True)

        @pl.when(g0 + 2 < nch)
        def _():
            pltpu.async_copy(table_hbm.at[srcv2.at[g0 + 2]], rows0, sem0)

        pltpu.make_async_copy(table_hbm.at[srcv2.at[g1]], rows1, sem1).wait()
        pltpu.sync_copy(rows1, accs.at[dstv2.at[g1]], add=True)

        @pl.when(g1 + 2 < nch)
        def _():
            pltpu.async_copy(table_hbm.at[srcv2.at[g1 + 2]], rows1, sem1)

        return carry

    lax.fori_loop(0, nch // 2, body, 0)


def _msum_blocks(table_hbm, accs, src2_hbm, dst2_hbm, srcv2, dstv2,
                 rows0, rows1, sem0, sem1, tile0, nch):
    def blk(b, carry):
        boff = pl.multiple_of(tile0 + b * _BI, 8)
        pltpu.sync_copy(src2_hbm.at[pl.ds(boff, _BI)], srcv2)
        pltpu.sync_copy(dst2_hbm.at[pl.ds(boff, _BI)], dstv2)
        _msum_pipeline(table_hbm, accs, srcv2, dstv2, rows0, rows1,
                       sem0, sem1, _BI)
        return carry

    lax.fori_loop(0, nch // _BI, blk, 0)


def _sc_msum0_body(x_hbm, src2_hbm, dst2_hbm, z128_hbm, msum_hbm,
                   srcv2, dstv2, rows0, rows1, accs, sem0, sem1):
    cid = lax.axis_index("c")
    sid = lax.axis_index("s")
    wid = cid * _NS + sid
    nch = E // _KE // _TILES
    pltpu.sync_copy(z128_hbm, accs.at[_trows(sid)])
    plsc.subcore_barrier()
    _msum_blocks(x_hbm, accs, src2_hbm, dst2_hbm, srcv2, dstv2,
                 rows0, rows1, sem0, sem1, wid * nch, nch)
    plsc.subcore_barrier()

    @pl.when(cid == 0)
    def _():
        pltpu.sync_copy(accs.at[_trows(sid)], msum_hbm.at[0, _trows(sid)])

    @pl.when(cid == 1)
    def _():
        pltpu.sync_copy(accs.at[_trows(sid)], msum_hbm.at[1, _trows(sid)])


def _seg_msum0(x, src2, dst2):
    z128 = jnp.zeros((_RPT, D), jnp.float32)
    nch = E // _KE // _TILES
    call = functools.partial(
        pl.kernel,
        mesh=_sc_mesh(),
        out_type=jax.ShapeDtypeStruct((2, _NPAD, D), jnp.float32),
        scratch_types=[
            pltpu.VMEM((_BI, _KE), jnp.int32),
            pltpu.VMEM((_BI, _KE), jnp.int32),
            pltpu.VMEM((_KE, D), jnp.float32),
            pltpu.VMEM((_KE, D), jnp.float32),
            pltpu.VMEM_SHARED((_NPAD, D), jnp.float32),
            pltpu.SemaphoreType.DMA,
            pltpu.SemaphoreType.DMA,
        ],
    )(_sc_msum0_body)
    return call(x, src2, dst2, z128)


def _sc_deg_body(dst2_hbm, ones_hbm, z128_hbm, deg_hbm, dstv2, onesv, accs,
                 sem):
    cid = lax.axis_index("c")
    sid = lax.axis_index("s")
    wid = cid * _NS + sid
    nch = E // _KE // _TILES
    pltpu.sync_copy(z128_hbm, accs.at[_trows(sid)])
    pltpu.sync_copy(ones_hbm, onesv)
    plsc.subcore_barrier()

    def blk(b, carry):
        boff = pl.multiple_of(wid * nch + b * _BI, 8)
        pltpu.sync_copy(dst2_hbm.at[pl.ds(boff, _BI)], dstv2)

        def body(g, c2):
            pltpu.sync_copy(onesv, accs.at[dstv2.at[g]], add=True)
            return c2

        lax.fori_loop(0, _BI, body, 0)
        return carry

    lax.fori_loop(0, nch // _BI, blk, 0)
    plsc.subcore_barrier()

    @pl.when(cid == 0)
    def _():
        pltpu.sync_copy(accs.at[_trows(sid)], deg_hbm.at[0, _trows(sid)])

    @pl.when(cid == 1)
    def _():
        pltpu.sync_copy(accs.at[_trows(sid)], deg_hbm.at[1, _trows(sid)])


def _seg_deg(dst2):
    ones = jnp.ones((_KE, D), jnp.float32)
    z128 = jnp.zeros((_RPT, D), jnp.float32)
    nch = E // _KE // _TILES
    call = functools.partial(
        pl.kernel,
        mesh=_sc_mesh(),
        out_type=jax.ShapeDtypeStruct((2, _NPAD, D), jnp.float32),
        scratch_types=[
            pltpu.VMEM((_BI, _KE), jnp.int32),
            pltpu.VMEM((_KE, D), jnp.float32),
            pltpu.VMEM_SHARED((_NPAD, D), jnp.float32),
            pltpu.SemaphoreType.DMA,
        ],
    )(_sc_deg_body)
    return call(dst2, ones, z128)


def _sc_msum_h_body(ha_hbm, hb_hbm, src2_hbm, dst2_hbm, z128_hbm, out_hbm,
                    srcv2, dstv2, rows0, rows1, accs, sem0, sem1):
    cid = lax.axis_index("c")
    sid = lax.axis_index("s")
    nch = E // _KE // _NS
    pltpu.sync_copy(z128_hbm, accs.at[_trows(sid)])
    plsc.subcore_barrier()

    @pl.when(cid == 0)
    def _():
        _msum_blocks(ha_hbm, accs, src2_hbm, dst2_hbm, srcv2, dstv2,
                     rows0, rows1, sem0, sem1, sid * nch, nch)

    @pl.when(cid == 1)
    def _():
        _msum_blocks(hb_hbm, accs, src2_hbm, dst2_hbm, srcv2, dstv2,
                     rows0, rows1, sem0, sem1, sid * nch, nch)

    plsc.subcore_barrier()

    @pl.when(cid == 0)
    def _():
        pltpu.sync_copy(accs.at[_trows(sid)], out_hbm.at[0, _trows(sid)])

    @pl.when(cid == 1)
    def _():
        pltpu.sync_copy(accs.at[_trows(sid)], out_hbm.at[1, _trows(sid)])


def _seg_msum_h(h, src2, dst2):
    z128 = jnp.zeros((_RPT, D), jnp.float32)
    call = functools.partial(
        pl.kernel,
        mesh=_sc_mesh(),
        out_type=jax.ShapeDtypeStruct((2, _NPAD, D), jnp.float32),
        scratch_types=[
            pltpu.VMEM((_BI, _KE), jnp.int32),
            pltpu.VMEM((_BI, _KE), jnp.int32),
            pltpu.VMEM((_KE, D), jnp.float32),
            pltpu.VMEM((_KE, D), jnp.float32),
            pltpu.VMEM_SHARED((_NPAD, D), jnp.float32),
            pltpu.SemaphoreType.DMA,
            pltpu.SemaphoreType.DMA,
        ],
    )(_sc_msum_h_body)
    return call(h[0], h[1], src2, dst2, z128)


def _sc_contact_body(z_hbm, ci2_hbm, cj2_hbm, zi_hbm, zj_hbm, iv2, jv2,
                     ri0, ri1, ri2, ri3, rj0, rj1, rj2, rj3,
                     si0, si1, si2, si3, sj0, sj1, sj2, sj3,
                     wi0, wi1, wi2, wi3, wj0, wj1, wj2, wj3):
    cid = lax.axis_index("c")
    sid = lax.axis_index("s")
    wid = cid * _NS + sid
    nch = EPP // _KC // _TILES
    pltpu.sync_copy(ci2_hbm.at[pl.ds(pl.multiple_of(wid * nch, 8), nch)],
                    iv2)
    pltpu.sync_copy(cj2_hbm.at[pl.ds(pl.multiple_of(wid * nch, 8), nch)],
                    jv2)
    ri = [ri0, ri1, ri2, ri3]
    rj = [rj0, rj1, rj2, rj3]
    si = [si0, si1, si2, si3]
    sj = [sj0, sj1, sj2, sj3]
    wi = [wi0, wi1, wi2, wi3]
    wj = [wj0, wj1, wj2, wj3]

    def obase(c):
        return pl.multiple_of((wid * nch + c) * _KC, 8)

    def wait_write(b, c):
        pltpu.make_async_copy(ri[b], zi_hbm.at[pl.ds(obase(c), _KC)],
                              wi[b]).wait()
        pltpu.make_async_copy(rj[b], zj_hbm.at[pl.ds(obase(c), _KC)],
                              wj[b]).wait()

    def start_gather(b, p):
        pltpu.async_copy(z_hbm.at[iv2.at[p]], ri[b], si[b])
        pltpu.async_copy(z_hbm.at[jv2.at[p]], rj[b], sj[b])

    def retire(b, c):
        pltpu.make_async_copy(z_hbm.at[iv2.at[c]], ri[b], si[b]).wait()
        pltpu.async_copy(ri[b], zi_hbm.at[pl.ds(obase(c), _KC)], wi[b])
        pltpu.make_async_copy(z_hbm.at[jv2.at[c]], rj[b], sj[b]).wait()
        pltpu.async_copy(rj[b], zj_hbm.at[pl.ds(obase(c), _KC)], wj[b])

    def round_(k, p):
        @pl.when(p >= 4)
        def _():
            wait_write(k, p - 4)

        start_gather(k, p)

        @pl.when(p >= 2)
        def _():
            retire((k + 2) % 4, p - 2)

    def body(t, carry):
        for k in range(4):
            round_(k, t * 4 + k)
        return carry

    lax.fori_loop(0, nch // 4, body, 0)
    retire(2, nch - 2)
    retire(3, nch - 1)
    for b in range(4):
        wait_write(b, nch - 4 + b)


def _contact_gather(z, ci2, cj2):
    nch = EPP // _KC // _TILES
    call = functools.partial(
        pl.kernel,
        mesh=_sc_mesh(),
        out_type=[jax.ShapeDtypeStruct((EPP, D), jnp.float32),
                  jax.ShapeDtypeStruct((EPP, D), jnp.float32)],
        scratch_types=(
            [pltpu.VMEM((nch, _KC), jnp.int32)] * 2
            + [pltpu.VMEM((_KC, D), jnp.float32)] * 8
            + [pltpu.SemaphoreType.DMA] * 16
        ),
    )(_sc_contact_body)
    return call(z, ci2, cj2)


def kernel(x, edge_index, contact_pred_index, Wl0, bl0, Wr0, g0, be0, ms0,
           Wl1, bl1, Wr1, g1, be1, ms1, Wl2, bl2, Wr2, g2, be2, ms2, alpha,
           dw, db, W1, b1, W2, b2, W3, b3, C1, cb1, C2, cb2, C3, cb3):
    src2 = edge_index[0].reshape(E // _KE, _KE)
    dst2 = edge_index[1].reshape(E // _KE, _KE)
    ci2 = jnp.pad(contact_pred_index[0], (0, EPP - EP)).reshape(
        EPP // _KC, _KC)
    cj2 = jnp.pad(contact_pred_index[1], (0, EPP - EP)).reshape(
        EPP // _KC, _KC)

    msum0 = _seg_msum0(x, src2, dst2)
    deg = _seg_deg(dst2)
    y0, st0 = _layer0(msum0, deg, x, Wl0, bl0, Wr0)
    h0 = _norm(y0, st0, g0, be0, ms0)

    msum1 = _seg_msum_h(h0, src2, dst2)
    y1, st1 = _layer(msum1, deg, h0, Wl1, bl1, Wr1)
    h1 = _norm(y1, st1, g1, be1, ms1, res=h0)

    msum2 = _seg_msum_h(h1, src2, dst2)
    y2, st2 = _layer(msum2, deg, h1, Wl2, bl2, Wr2)

    z = _zstage(y2, st2, g2, be2, ms2, h1, h0, x, alpha, dw, db,
                W1, b1, W2, b2, W3, b3)

    zi, zj = _contact_gather(z, ci2, cj2)
    probs = _head(zi, zj, C1, cb1, C2, cb2, C3, cb3)
    return probs[:EP]


# speedup vs baseline: 1.0376x; 1.0376x over previous
import functools

import jax
import jax.numpy as jnp
from jax import lax
from jax.experimental import pallas as pl
from jax.experimental.pallas import tpu as pltpu
from jax.experimental.pallas import tpu_sc as plsc

N = 10000
E = 320000
EP = 200000
D = 128
H = 256
R = 400
EPP = 204800
RC = 2048

_SQRT_HALF = 0.7071067811865476


def _matT(a, b):
    return lax.dot_general(a, b, (((1,), (1,)), ((), ())),
                           preferred_element_type=jnp.float32)


def _gelu(v):
    return 0.5 * v * (1.0 + lax.erf(v * _SQRT_HALF))


def _a0_body(msum_ref, deg_ref, x_ref, wl_ref, bl_ref, wr_ref, y_ref, st_ref):
    i = pl.program_id(0)
    rdc = 1.0 / jnp.maximum(deg_ref[0, :, 0:1] + deg_ref[1, :, 0:1], 1.0)
    mean = (msum_ref[0] + msum_ref[1]) * rdc
    y = _matT(mean, wl_ref[...]) + bl_ref[...] + _matT(x_ref[...], wr_ref[...])
    y = _gelu(y)
    y_ref[...] = y

    @pl.when(i == 0)
    def _():
        st_ref[...] = jnp.zeros_like(st_ref)

    st_ref[0:1, :] += jnp.sum(y, axis=0, keepdims=True)
    st_ref[1:2, :] += jnp.sum(y * y, axis=0, keepdims=True)


def _a1_body(msum_ref, deg_ref, h_ref, wla_ref, wlb_ref, bl_ref, wra_ref,
             wrb_ref, y_ref, st_ref):
    i = pl.program_id(0)
    rdc = 1.0 / jnp.maximum(deg_ref[0, :, 0:1] + deg_ref[1, :, 0:1], 1.0)
    y = (_matT(msum_ref[0] * rdc, wla_ref[...])
         + _matT(msum_ref[1] * rdc, wlb_ref[...])
         + bl_ref[...]
         + _matT(h_ref[0], wra_ref[...])
         + _matT(h_ref[1], wrb_ref[...]))
    y = _gelu(y)
    y_ref[...] = y

    @pl.when(i == 0)
    def _():
        st_ref[...] = jnp.zeros_like(st_ref)

    st_ref[0:1, :] += jnp.sum(y, axis=0, keepdims=True)
    st_ref[1:2, :] += jnp.sum(y * y, axis=0, keepdims=True)


def _norm0_body(y_ref, st_ref, g_ref, be_ref, ms_ref, h_ref):
    mu = st_ref[0:1, :] * (1.0 / N)
    m2 = st_ref[1:2, :] * (1.0 / N)
    c = mu * ms_ref[...]
    var = m2 - 2.0 * c * mu + c * c
    scale = g_ref[...] * lax.rsqrt(var + 1e-5)
    h = (y_ref[...] - c) * scale + be_ref[...]
    h_ref[0] = h[:, :D]
    h_ref[1] = h[:, D:]


def _norm_res_body(y_ref, st_ref, g_ref, be_ref, ms_ref, r_ref, h_ref):
    mu = st_ref[0:1, :] * (1.0 / N)
    m2 = st_ref[1:2, :] * (1.0 / N)
    c = mu * ms_ref[...]
    var = m2 - 2.0 * c * mu + c * c
    scale = g_ref[...] * lax.rsqrt(var + 1e-5)
    h = (y_ref[...] - c) * scale + be_ref[...]
    h_ref[0] = h[:, :D] + r_ref[0]
    h_ref[1] = h[:, D:] + r_ref[1]


def _z_body(y2_ref, st_ref, g_ref, be_ref, ms_ref, h1_ref, h0_ref, x_ref,
            alpha_ref, dw_ref, db_ref, w1_ref, b1_ref, w2_ref, b2_ref,
            w3_ref, b3_ref, z_ref):
    mu = st_ref[0:1, :] * (1.0 / N)
    m2 = st_ref[1:2, :] * (1.0 / N)
    c = mu * ms_ref[...]
    var = m2 - 2.0 * c * mu + c * c
    scale = g_ref[...] * lax.rsqrt(var + 1e-5)
    h1 = jnp.concatenate([h1_ref[0], h1_ref[1]], axis=1)
    h0 = jnp.concatenate([h0_ref[0], h0_ref[1]], axis=1)
    h2 = (y2_ref[...] - c) * scale + be_ref[...] + h1
    z = jnp.concatenate([h0, h1, h2], axis=1)
    z = jnp.tanh(alpha_ref[0, 0] * z) * dw_ref[...] + db_ref[...]
    t = _gelu(_matT(z, w1_ref[...]) + b1_ref[...])
    t = _gelu(_matT(t, w2_ref[...]) + b2_ref[...])
    t = _matT(t, w3_ref[...]) + b3_ref[...]
    t = t + x_ref[...]
    nrm = jnp.sqrt(jnp.sum(t * t, axis=1, keepdims=True))
    z_ref[...] = t / (nrm + 1e-10)


def _head_body(zi_ref, zj_ref, c1a_ref, c1b_ref, cb1_ref, c2_ref, cb2_ref,
               c3_ref, cb3_ref, o_ref):
    w = _gelu(_matT(zi_ref[...], c1a_ref[...]) + _matT(zj_ref[...], c1b_ref[...])
              + cb1_ref[...])
    w = _gelu(_matT(w, c2_ref[...]) + cb2_ref[...])
    s = jnp.sum(w * c3_ref[...], axis=1) + cb3_ref[0, 0]
    o_ref[...] = 1.0 / (1.0 + jnp.exp(-s))


_GRID = N // R


def _spec(shape, idx):
    return pl.BlockSpec(shape, idx)


def _row_specs():
    return dict(
        msum=_spec((2, R, D), lambda i: (0, i, 0)),
        deg=_spec((2, R, D), lambda i: (0, i, 0)),
        xrow=_spec((R, D), lambda i: (i, 0)),
        hrow=_spec((2, R, D), lambda i: (0, i, 0)),
        yrow=_spec((R, H), lambda i: (i, 0)),
        st=_spec((8, H), lambda i: (0, 0)),
    )


def _layer0(msum, deg, x, Wl0, bl0, Wr0):
    s = _row_specs()
    y, st = pl.pallas_call(
        _a0_body,
        grid=(_GRID,),
        in_specs=[s["msum"], s["deg"], s["xrow"],
                  _spec((H, D), lambda i: (0, 0)),
                  _spec((1, H), lambda i: (0, 0)),
                  _spec((H, D), lambda i: (0, 0))],
        out_specs=[s["yrow"], s["st"]],
        out_shape=[jax.ShapeDtypeStruct((N, H), jnp.float32),
                   jax.ShapeDtypeStruct((8, H), jnp.float32)],
    )(msum, deg, x, Wl0, bl0.reshape(1, H), Wr0)
    return y, st


def _layer(msum, deg, h, Wl, bl, Wr):
    s = _row_specs()
    wspec = _spec((H, D), lambda i: (0, 0))
    y, st = pl.pallas_call(
        _a1_body,
        grid=(_GRID,),
        in_specs=[s["msum"], s["deg"], s["hrow"], wspec, wspec,
                  _spec((1, H), lambda i: (0, 0)), wspec, wspec],
        out_specs=[s["yrow"], s["st"]],
        out_shape=[jax.ShapeDtypeStruct((N, H), jnp.float32),
                   jax.ShapeDtypeStruct((8, H), jnp.float32)],
    )(msum, deg, h, Wl[:, :D], Wl[:, D:], bl.reshape(1, H),
      Wr[:, :D], Wr[:, D:])
    return y, st


def _norm(y, st, g, be, ms, res=None):
    s = _row_specs()
    vspec = _spec((1, H), lambda i: (0, 0))
    in_specs = [s["yrow"], s["st"], vspec, vspec, vspec]
    args = [y, st, g.reshape(1, H), be.reshape(1, H), ms.reshape(1, H)]
    body = _norm0_body
    if res is not None:
        in_specs.append(s["hrow"])
        args.append(res)
        body = _norm_res_body
    return pl.pallas_call(
        body,
        grid=(_GRID,),
        in_specs=in_specs,
        out_specs=s["hrow"],
        out_shape=jax.ShapeDtypeStruct((2, _NPAD, D), jnp.float32),
    )(*args)


def _zstage(y2, st2, g2, be2, ms2, h1, h0, x, alpha, dw, db, W1, b1, W2, b2,
            W3, b3):
    s = _row_specs()
    vspec = _spec((1, H), lambda i: (0, 0))
    JK = 3 * H
    z = pl.pallas_call(
        _z_body,
        grid=(_GRID,),
        in_specs=[s["yrow"], s["st"], vspec, vspec, vspec, s["hrow"],
                  s["hrow"], s["xrow"],
                  _spec((1, 1), lambda i: (0, 0)),
                  _spec((1, JK), lambda i: (0, 0)),
                  _spec((1, JK), lambda i: (0, 0)),
                  _spec((H, JK), lambda i: (0, 0)),
                  _spec((1, H), lambda i: (0, 0)),
                  _spec((H, H), lambda i: (0, 0)),
                  _spec((1, H), lambda i: (0, 0)),
                  _spec((D, H), lambda i: (0, 0)),
                  _spec((1, D), lambda i: (0, 0))],
        out_specs=_spec((R, D), lambda i: (i, 0)),
        out_shape=jax.ShapeDtypeStruct((N, D), jnp.float32),
    )(y2, st2, g2.reshape(1, H), be2.reshape(1, H), ms2.reshape(1, H),
      h1, h0, x, alpha.reshape(1, 1), dw.reshape(1, JK), db.reshape(1, JK),
      W1, b1.reshape(1, H), W2, b2.reshape(1, H), W3, b3.reshape(1, D))
    return z


def _head(zi, zj, C1, cb1, C2, cb2, C3, cb3):
    return pl.pallas_call(
        _head_body,
        grid=(EPP // RC,),
        in_specs=[_spec((RC, D), lambda i: (i, 0)),
                  _spec((RC, D), lambda i: (i, 0)),
                  _spec((64, D), lambda i: (0, 0)),
                  _spec((64, D), lambda i: (0, 0)),
                  _spec((1, 64), lambda i: (0, 0)),
                  _spec((64, 64), lambda i: (0, 0)),
                  _spec((1, 64), lambda i: (0, 0)),
                  _spec((1, 64), lambda i: (0, 0)),
                  _spec((1, 1), lambda i: (0, 0))],
        out_specs=_spec((RC,), lambda i: (i,)),
        out_shape=jax.ShapeDtypeStruct((EPP,), jnp.float32),
    )(zi, zj, C1[:, :D], C1[:, D:], cb1.reshape(1, 64), C2,
      cb2.reshape(1, 64), C3.reshape(1, 64), cb3.reshape(1, 1))


_NC = 2
_NS = 16
_TILES = _NC * _NS
_KE = 125
_KC = 80
_BI = 32
_NPAD = 10240
_RPT = _NPAD // _NS


def _sc_mesh():
    return plsc.VectorSubcoreMesh(core_axis_name="c", subcore_axis_name="s")


def _trows(sid):
    return pl.ds(pl.multiple_of(sid * _RPT, 8), _RPT)


def _msum_pipeline(table_hbm, accs, srcv2, dstv2, rows0, rows1, sem0, sem1,
                   nch):
    pltpu.async_copy(table_hbm.at[srcv2.at[0]], rows0, sem0)
    pltpu.async_copy(table_hbm.at[srcv2.at[1]], rows1, sem1)

    def body(t, carry):
        g0 = t * 2
        g1 = g0 + 1
        pltpu.make_async_copy(table_hbm.at[srcv2.at[g0]], rows0, sem0).wait()
        pltpu.sync_copy(rows0, accs.at[dstv2.at[g0]], add=True)

        @pl.when(g0 + 2 < nch)
        def _():
            pltpu.async_copy(table_hbm.at[srcv2.at[g0 + 2]], rows0, sem0)

        pltpu.make_async_copy(table_hbm.at[srcv2.at[g1]], rows1, sem1).wait()
        pltpu.sync_copy(rows1, accs.at[dstv2.at[g1]], add=True)

        @pl.when(g1 + 2 < nch)
        def _():
            pltpu.async_copy(table_hbm.at[srcv2.at[g1 + 2]], rows1, sem1)

        return carry

    lax.fori_loop(0, nch // 2, body, 0)


def _msum_blocks(table_hbm, accs, src2_hbm, dst2_hbm, srcv2, dstv2,
                 rows0, rows1, sem0, sem1, tile0, nch):
    def blk(b, carry):
        boff = pl.multiple_of(tile0 + b * _BI, 8)
        pltpu.sync_copy(src2_hbm.at[pl.ds(boff, _BI)], srcv2)
        pltpu.sync_copy(dst2_hbm.at[pl.ds(boff, _BI)], dstv2)
        _msum_pipeline(table_hbm, accs, srcv2, dstv2, rows0, rows1,
                       sem0, sem1, _BI)
        return carry

    lax.fori_loop(0, nch // _BI, blk, 0)


def _sc_msum0_body(x_hbm, src2_hbm, dst2_hbm, z128_hbm, msum_hbm,
                   srcv2, dstv2, rows0, rows1, accs, sem0, sem1):
    cid = lax.axis_index("c")
    sid = lax.axis_index("s")
    wid = cid * _NS + sid
    nch = E // _KE // _TILES
    pltpu.sync_copy(z128_hbm, accs.at[_trows(sid)])
    plsc.subcore_barrier()
    _msum_blocks(x_hbm, accs, src2_hbm, dst2_hbm, srcv2, dstv2,
                 rows0, rows1, sem0, sem1, wid * nch, nch)
    plsc.subcore_barrier()

    @pl.when(cid == 0)
    def _():
        pltpu.sync_copy(accs.at[_trows(sid)], msum_hbm.at[0, _trows(sid)])

    @pl.when(cid == 1)
    def _():
        pltpu.sync_copy(accs.at[_trows(sid)], msum_hbm.at[1, _trows(sid)])


def _seg_msum0(x, src2, dst2):
    z128 = jnp.zeros((_RPT, D), jnp.float32)
    nch = E // _KE // _TILES
    call = functools.partial(
        pl.kernel,
        mesh=_sc_mesh(),
        out_type=jax.ShapeDtypeStruct((2, _NPAD, D), jnp.float32),
        scratch_types=[
            pltpu.VMEM((_BI, _KE), jnp.int32),
            pltpu.VMEM((_BI, _KE), jnp.int32),
            pltpu.VMEM((_KE, D), jnp.float32),
            pltpu.VMEM((_KE, D), jnp.float32),
            pltpu.VMEM_SHARED((_NPAD, D), jnp.float32),
            pltpu.SemaphoreType.DMA,
            pltpu.SemaphoreType.DMA,
        ],
    )(_sc_msum0_body)
    return call(x, src2, dst2, z128)


def _sc_deg_body(dst2_hbm, ones_hbm, z128_hbm, deg_hbm, dstv2, onesv, accs,
                 sem):
    cid = lax.axis_index("c")
    sid = lax.axis_index("s")
    wid = cid * _NS + sid
    nch = E // _KE // _TILES
    pltpu.sync_copy(z128_hbm, accs.at[_trows(sid)])
    pltpu.sync_copy(ones_hbm, onesv)
    plsc.subcore_barrier()

    def blk(b, carry):
        boff = pl.multiple_of(wid * nch + b * _BI, 8)
        pltpu.sync_copy(dst2_hbm.at[pl.ds(boff, _BI)], dstv2)

        def body(g, c2):
            pltpu.sync_copy(onesv, accs.at[dstv2.at[g]], add=True)
            return c2

        lax.fori_loop(0, _BI, body, 0)
        return carry

    lax.fori_loop(0, nch // _BI, blk, 0)
    plsc.subcore_barrier()

    @pl.when(cid == 0)
    def _():
        pltpu.sync_copy(accs.at[_trows(sid)], deg_hbm.at[0, _trows(sid)])

    @pl.when(cid == 1)
    def _():
        pltpu.sync_copy(accs.at[_trows(sid)], deg_hbm.at[1, _trows(sid)])


def _seg_deg(dst2):
    ones = jnp.ones((_KE, D), jnp.float32)
    z128 = jnp.zeros((_RPT, D), jnp.float32)
    nch = E // _KE // _TILES
    call = functools.partial(
        pl.kernel,
        mesh=_sc_mesh(),
        out_type=jax.ShapeDtypeStruct((2, _NPAD, D), jnp.float32),
        scratch_types=[
            pltpu.VMEM((_BI, _KE), jnp.int32),
            pltpu.VMEM((_KE, D), jnp.float32),
            pltpu.VMEM_SHARED((_NPAD, D), jnp.float32),
            pltpu.SemaphoreType.DMA,
        ],
    )(_sc_deg_body)
    return call(dst2, ones, z128)


def _sc_msum_h_body(ha_hbm, hb_hbm, src2_hbm, dst2_hbm, z128_hbm, out_hbm,
                    srcv2, dstv2, rows0, rows1, accs, sem0, sem1):
    cid = lax.axis_index("c")
    sid = lax.axis_index("s")
    nch = E // _KE // _NS
    pltpu.sync_copy(z128_hbm, accs.at[_trows(sid)])
    plsc.subcore_barrier()

    @pl.when(cid == 0)
    def _():
        _msum_blocks(ha_hbm, accs, src2_hbm, dst2_hbm, srcv2, dstv2,
                     rows0, rows1, sem0, sem1, sid * nch, nch)

    @pl.when(cid == 1)
    def _():
        _msum_blocks(hb_hbm, accs, src2_hbm, dst2_hbm, srcv2, dstv2,
                     rows0, rows1, sem0, sem1, sid * nch, nch)

    plsc.subcore_barrier()

    @pl.when(cid == 0)
    def _():
        pltpu.sync_copy(accs.at[_trows(sid)], out_hbm.at[0, _trows(sid)])

    @pl.when(cid == 1)
    def _():
        pltpu.sync_copy(accs.at[_trows(sid)], out_hbm.at[1, _trows(sid)])


def _seg_msum_h(h, src2, dst2):
    z128 = jnp.zeros((_RPT, D), jnp.float32)
    call = functools.partial(
        pl.kernel,
        mesh=_sc_mesh(),
        out_type=jax.ShapeDtypeStruct((2, _NPAD, D), jnp.float32),
        scratch_types=[
            pltpu.VMEM((_BI, _KE), jnp.int32),
            pltpu.VMEM((_BI, _KE), jnp.int32),
            pltpu.VMEM((_KE, D), jnp.float32),
            pltpu.VMEM((_KE, D), jnp.float32),
            pltpu.VMEM_SHARED((_NPAD, D), jnp.float32),
            pltpu.SemaphoreType.DMA,
            pltpu.SemaphoreType.DMA,
        ],
    )(_sc_msum_h_body)
    return call(h[0], h[1], src2, dst2, z128)


def _sc_contact_body(z_hbm, ci2_hbm, cj2_hbm, zi_hbm, zj_hbm, iv2, jv2,
                     ri0, ri1, ri2, ri3, rj0, rj1, rj2, rj3,
                     si0, si1, si2, si3, sj0, sj1, sj2, sj3,
                     wi0, wi1, wi2, wi3, wj0, wj1, wj2, wj3):
    cid = lax.axis_index("c")
    sid = lax.axis_index("s")
    wid = cid * _NS + sid
    nch = EPP // _KC // _TILES
    pltpu.sync_copy(ci2_hbm.at[pl.ds(pl.multiple_of(wid * nch, 8), nch)],
                    iv2)
    pltpu.sync_copy(cj2_hbm.at[pl.ds(pl.multiple_of(wid * nch, 8), nch)],
                    jv2)
    ri = [ri0, ri1, ri2, ri3]
    rj = [rj0, rj1, rj2, rj3]
    si = [si0, si1, si2, si3]
    sj = [sj0, sj1, sj2, sj3]
    wi = [wi0, wi1, wi2, wi3]
    wj = [wj0, wj1, wj2, wj3]

    def obase(c):
        return pl.multiple_of((wid * nch + c) * _KC, 8)

    def wait_write(b, c):
        pltpu.make_async_copy(ri[b], zi_hbm.at[pl.ds(obase(c), _KC)],
                              wi[b]).wait()
        pltpu.make_async_copy(rj[b], zj_hbm.at[pl.ds(obase(c), _KC)],
                              wj[b]).wait()

    def start_gather(b, p):
        pltpu.async_copy(z_hbm.at[iv2.at[p]], ri[b], si[b])
        pltpu.async_copy(z_hbm.at[jv2.at[p]], rj[b], sj[b])

    def retire(b, c):
        pltpu.make_async_copy(z_hbm.at[iv2.at[c]], ri[b], si[b]).wait()
        pltpu.async_copy(ri[b], zi_hbm.at[pl.ds(obase(c), _KC)], wi[b])
        pltpu.make_async_copy(z_hbm.at[jv2.at[c]], rj[b], sj[b]).wait()
        pltpu.async_copy(rj[b], zj_hbm.at[pl.ds(obase(c), _KC)], wj[b])

    def round_(k, p):
        @pl.when(p >= 4)
        def _():
            wait_write(k, p - 4)

        start_gather(k, p)

        @pl.when(p >= 2)
        def _():
            retire((k + 2) % 4, p - 2)

    def body(t, carry):
        for k in range(4):
            round_(k, t * 4 + k)
        return carry

    lax.fori_loop(0, nch // 4, body, 0)
    retire(2, nch - 2)
    retire(3, nch - 1)
    for b in range(4):
        wait_write(b, nch - 4 + b)


def _contact_gather(z, ci2, cj2):
    nch = EPP // _KC // _TILES
    call = functools.partial(
        pl.kernel,
        mesh=_sc_mesh(),
        out_type=[jax.ShapeDtypeStruct((EPP, D), jnp.float32),
                  jax.ShapeDtypeStruct((EPP, D), jnp.float32)],
        scratch_types=(
            [pltpu.VMEM((nch, _KC), jnp.int32)] * 2
            + [pltpu.VMEM((_KC, D), jnp.float32)] * 8
            + [pltpu.SemaphoreType.DMA] * 16
        ),
    )(_sc_contact_body)
    return call(z, ci2, cj2)


def kernel(x, edge_index, contact_pred_index, Wl0, bl0, Wr0, g0, be0, ms0,
           Wl1, bl1, Wr1, g1, be1, ms1, Wl2, bl2, Wr2, g2, be2, ms2, alpha,
           dw, db, W1, b1, W2, b2, W3, b3, C1, cb1, C2, cb2, C3, cb3):
    src2 = edge_index[0].reshape(E // _KE, _KE)
    dst2 = edge_index[1].reshape(E // _KE, _KE)
    ci2 = jnp.pad(contact_pred_index[0], (0, EPP - EP)).reshape(
        EPP // _KC, _KC)
    cj2 = jnp.pad(contact_pred_index[1], (0, EPP - EP)).reshape(
        EPP // _KC, _KC)

    msum0 = _seg_msum0(x, src2, dst2)
    deg = _seg_deg(dst2)
    y0, st0 = _layer0(msum0, deg, x, Wl0, bl0, Wr0)
    h0 = _norm(y0, st0, g0, be0, ms0)

    msum1 = _seg_msum_h(h0, src2, dst2)
    y1, st1 = _layer(msum1, deg, h0, Wl1, bl1, Wr1)
    h1 = _norm(y1, st1, g1, be1, ms1, res=h0)

    msum2 = _seg_msum_h(h1, src2, dst2)
    y2, st2 = _layer(msum2, deg, h1, Wl2, bl2, Wr2)

    z = _zstage(y2, st2, g2, be2, ms2, h1, h0, x, alpha, dw, db,
                W1, b1, W2, b2, W3, b3)

    zi, zj = _contact_gather(z, ci2, cj2)
    probs = _head(zi, zj, C1, cb1, C2, cb2, C3, cb3)
    return probs[:EP]
